# Initial kernel scaffold; baseline (speedup 1.0000x reference)
import jax
import jax.numpy as jnp
from jax.experimental import pallas as pl


def kernel(input_feature, adj, W1, a1, W_out, a_out):
    raise NotImplementedError("write your pallas kernel here")


# speedup vs baseline: 2.2970x; 2.2970x over previous
import jax
import jax.numpy as jnp
from jax.experimental import pallas as pl

_B, _N, _IN, _OUT, _H = 8, 1024, 128, 64, 4
_NEG = -9e15


def _fused_gat_kernel(x_ref, adj_ref, w1_ref, a1s_ref, a1d_ref, wout_ref,
                      aout_ref, out_ref):
    x = x_ref[0]
    mask = jnp.where(adj_ref[0] > 0, 0.0, _NEG).astype(jnp.float32)

    wh = jnp.dot(x, w1_ref[...], preferred_element_type=jnp.float32)
    es = jnp.dot(wh, a1s_ref[...], preferred_element_type=jnp.float32)
    ed = jnp.dot(wh, a1d_ref[...], preferred_element_type=jnp.float32)
    edt = ed.T

    acc = jnp.zeros((_N, _OUT), jnp.float32)
    for h in range(_H):
        s = es[:, h:h + 1] + edt[h:h + 1, :]
        e = jnp.maximum(s, 0.2 * s) + mask
        m = jnp.max(e, axis=1, keepdims=True)
        p = jnp.exp(e - m)
        r = 1.0 / jnp.sum(p, axis=1, keepdims=True)
        hp = jnp.dot(p, wh[:, h * _OUT:(h + 1) * _OUT],
                     preferred_element_type=jnp.float32) * r
        acc = acc + jnp.where(hp > 0, hp, jnp.exp(hp) - 1.0)

    x2 = acc * (1.0 / _H)
    wh2 = jnp.dot(x2, wout_ref[...], preferred_element_type=jnp.float32)
    e2 = jnp.dot(wh2, aout_ref[...], preferred_element_type=jnp.float32)
    e2t = e2.T
    s = e2[:, 0:1] + e2t[1:2, :]
    e = jnp.maximum(s, 0.2 * s) + mask
    m = jnp.max(e, axis=1, keepdims=True)
    p = jnp.exp(e - m)
    r = 1.0 / jnp.sum(p, axis=1, keepdims=True)
    o = jnp.dot(p, wh2, preferred_element_type=jnp.float32) * r
    out_ref[0] = jnp.maximum(o, 0.0)


def kernel(input_feature, adj, W1, a1, W_out, a_out):
    w1r = jnp.transpose(W1, (1, 0, 2)).reshape(_IN, _H * _OUT)
    a_src = a1[:, :_OUT, 0]
    a_dst = a1[:, _OUT:, 0]
    eye = jnp.eye(_H, dtype=jnp.float32)
    a1s = (eye[:, None, :] * a_src[:, :, None]).reshape(_H * _OUT, _H)
    a1d = (eye[:, None, :] * a_dst[:, :, None]).reshape(_H * _OUT, _H)
    aout2 = a_out.reshape(2, _OUT).T

    grid_spec = pl.GridSpec(
        grid=(_B,),
        in_specs=[
            pl.BlockSpec((1, _N, _IN), lambda b: (b, 0, 0)),
            pl.BlockSpec((1, _N, _N), lambda b: (b, 0, 0)),
            pl.BlockSpec((_IN, _H * _OUT), lambda b: (0, 0)),
            pl.BlockSpec((_H * _OUT, _H), lambda b: (0, 0)),
            pl.BlockSpec((_H * _OUT, _H), lambda b: (0, 0)),
            pl.BlockSpec((_OUT, _OUT), lambda b: (0, 0)),
            pl.BlockSpec((_OUT, 2), lambda b: (0, 0)),
        ],
        out_specs=pl.BlockSpec((1, _N, _OUT), lambda b: (b, 0, 0)),
    )
    return pl.pallas_call(
        _fused_gat_kernel,
        grid_spec=grid_spec,
        out_shape=jax.ShapeDtypeStruct((_B, _N, _OUT), jnp.float32),
    )(input_feature, adj, w1r, a1s, a1d, W_out, aout2)
